# baseline (speedup 1.0000x reference)
import functools

import jax
import jax.numpy as jnp
from jax import lax
from jax.experimental import pallas as pl
from jax.experimental.pallas import tpu as pltpu
from jax.experimental.pallas import tpu_sc as plsc

_N = 10000
_F = 128
_NCLS = 40
_NC = 2
_NS = 16
_NW = _NC * _NS
_LPW = _F // _NW
_E0 = 320000
_EP = 327680
_EC = 4096
_NCHUNK = _EP // _EC
_TRASH = _N
_NPL = 10016
_NFP = _NPL * _LPW

_ALPHA = 0.1
_RW = 0.1
_CS = 0.8

_mesh = plsc.VectorSubcoreMesh(core_axis_name="c", subcore_axis_name="s")


@functools.partial(
    pl.kernel,
    mesh=_mesh,
    compiler_params=pltpu.CompilerParams(needs_layout_passes=False),
    out_type=pltpu.HBM((_NW, _NFP), jnp.float32),
    scratch_types=[
        pltpu.VMEM((_NFP,), jnp.float32),
        pltpu.VMEM((_NFP,), jnp.float32),
        [pltpu.VMEM((_EC,), jnp.int32)] * 2,
        [pltpu.VMEM((_EC,), jnp.int32)] * 2,
        [pltpu.SemaphoreType.DMA] * 2,
        [pltpu.SemaphoreType.DMA] * 2,
        pltpu.SemaphoreType.DMA,
    ],
)
def _sc_aggregate(row_hbm, col_hbm, gt_hbm, out, g_t, acc_t,
                  ridxb, cidxb, rsem, csem, gsem):
    c = lax.axis_index("c")
    s = lax.axis_index("s")
    wid = s * _NC + c

    pltpu.async_copy(gt_hbm.at[wid], g_t, gsem)
    pltpu.async_copy(row_hbm.at[pl.ds(0, _EC)], ridxb[0], rsem[0])
    pltpu.async_copy(col_hbm.at[pl.ds(0, _EC)], cidxb[0], csem[0])

    zero16 = jnp.zeros((16,), jnp.float32)

    def z(i, carry):
        acc_t[pl.ds(i * 16, 16)] = zero16
        return carry

    lax.fori_loop(0, _NFP // 16, z, 0, unroll=4)
    pltpu.make_async_copy(gt_hbm.at[wid], g_t, gsem).wait()

    def pair(k, carry):
        for p in range(2):
            ch = k * 2 + p
            pltpu.make_async_copy(row_hbm.at[pl.ds(0, _EC)], ridxb[p], rsem[p]).wait()
            pltpu.make_async_copy(col_hbm.at[pl.ds(0, _EC)], cidxb[p], csem[p]).wait()

            @pl.when(ch + 1 < _NCHUNK)
            def _():
                nb = (ch + 1) * _EC
                pltpu.async_copy(row_hbm.at[pl.ds(nb, _EC)], ridxb[1 - p], rsem[1 - p])
                pltpu.async_copy(col_hbm.at[pl.ds(nb, _EC)], cidxb[1 - p], csem[1 - p])

            @plsc.parallel_loop(0, _EC // 16, unroll=8)
            def _(i):
                r16 = ridxb[p][pl.ds(i * 16, 16)]
                c16 = cidxb[p][pl.ds(i * 16, 16)]
                for l in range(_LPW):
                    v = plsc.load_gather(g_t, [r16 + (l * _NPL)])
                    plsc.addupdate_scatter(acc_t, [c16 + (l * _NPL)], v)
        return carry

    lax.fori_loop(0, _NCHUNK // 2, pair, 0)
    pltpu.sync_copy(acc_t, out.at[wid])


def _to_sc(g):
    gt = g.reshape(_N, _NW, _LPW).transpose(1, 2, 0)
    gt = jnp.pad(gt, ((0, 0), (0, 0), (0, _NPL - _N)))
    return gt.reshape(_NW, _NFP)


def _from_sc(p):
    return p.reshape(_NW, _LPW, _NPL)[:, :, :_N].transpose(2, 0, 1).reshape(_N, _F)


_BLK = 1000


def _tc_init_body(x_ref, win_ref, bin_ref, cnt_ref, h0_ref, g0_ref, dv_ref):
    z = jnp.dot(x_ref[...], win_ref[...], preferred_element_type=jnp.float32)
    h0 = jnp.maximum(z + bin_ref[...], 0.0)
    dv = lax.rsqrt(cnt_ref[...] + 1.0)
    h0_ref[...] = h0
    g0_ref[...] = h0 * dv
    dv_ref[...] = dv


def _tc_init(x, Win, bin_row, cnt):
    return pl.pallas_call(
        _tc_init_body,
        grid=(_N // _BLK,),
        in_specs=[
            pl.BlockSpec((_BLK, _F), lambda i: (i, 0)),
            pl.BlockSpec((_F, _F), lambda i: (0, 0)),
            pl.BlockSpec((1, _F), lambda i: (0, 0)),
            pl.BlockSpec((_BLK, 1), lambda i: (i, 0)),
        ],
        out_specs=[
            pl.BlockSpec((_BLK, _F), lambda i: (i, 0)),
            pl.BlockSpec((_BLK, _F), lambda i: (i, 0)),
            pl.BlockSpec((_BLK, 1), lambda i: (i, 0)),
        ],
        out_shape=[
            jax.ShapeDtypeStruct((_N, _F), jnp.float32),
            jax.ShapeDtypeStruct((_N, _F), jnp.float32),
            jax.ShapeDtypeStruct((_N, 1), jnp.float32),
        ],
    )(x, Win, bin_row, cnt)


def _tc_layer_body(p_ref, h_ref, x0_ref, dv_ref, w_ref, b_ref, hn_ref, gn_ref):
    dv = dv_ref[...]
    h = h_ref[...]
    agg = p_ref[...] * dv + (dv * dv) * h
    h2 = _CS * agg + _RW * h + _ALPHA * x0_ref[...]
    z = jnp.dot(h2, w_ref[...], preferred_element_type=jnp.float32)
    hn = jnp.maximum(z, b_ref[...])
    hn_ref[...] = hn
    gn_ref[...] = hn * dv


def _tc_layer(p, h, x0, dv, W, b_row):
    return pl.pallas_call(
        _tc_layer_body,
        grid=(_N // _BLK,),
        in_specs=[
            pl.BlockSpec((_BLK, _F), lambda i: (i, 0)),
            pl.BlockSpec((_BLK, _F), lambda i: (i, 0)),
            pl.BlockSpec((_BLK, _F), lambda i: (i, 0)),
            pl.BlockSpec((_BLK, 1), lambda i: (i, 0)),
            pl.BlockSpec((_F, _F), lambda i: (0, 0)),
            pl.BlockSpec((1, _F), lambda i: (0, 0)),
        ],
        out_specs=[
            pl.BlockSpec((_BLK, _F), lambda i: (i, 0)),
            pl.BlockSpec((_BLK, _F), lambda i: (i, 0)),
        ],
        out_shape=[
            jax.ShapeDtypeStruct((_N, _F), jnp.float32),
            jax.ShapeDtypeStruct((_N, _F), jnp.float32),
        ],
    )(p, h, x0, dv, W, b_row)


def _tc_out_body(h_ref, w_ref, b_ref, o_ref):
    o_ref[...] = (
        jnp.dot(h_ref[...], w_ref[...], preferred_element_type=jnp.float32)
        + b_ref[...]
    )


def _tc_out(h, Wout, bout_row):
    return pl.pallas_call(
        _tc_out_body,
        grid=(_N // _BLK,),
        in_specs=[
            pl.BlockSpec((_BLK, _F), lambda i: (i, 0)),
            pl.BlockSpec((_F, _NCLS), lambda i: (0, 0)),
            pl.BlockSpec((1, _NCLS), lambda i: (0, 0)),
        ],
        out_specs=pl.BlockSpec((_BLK, _NCLS), lambda i: (i, 0)),
        out_shape=jax.ShapeDtypeStruct((_N, _NCLS), jnp.float32),
    )(h, Wout, bout_row)


def kernel(x, edge_index, Win, bin_, Wg, srelu_bias, Wout, bout):
    npad = _EP - _E0
    row = jnp.concatenate([edge_index[0], jnp.zeros((npad,), jnp.int32)])
    col = jnp.concatenate([edge_index[1], jnp.full((npad,), _TRASH, jnp.int32)])

    onesT = jnp.ones((_NW, _NFP), jnp.float32)
    cntT = _sc_aggregate(row, col, onesT)
    cnt = cntT[0, 0:_N].reshape(_N, 1)

    h0, g, dv = _tc_init(x, Win, bin_.reshape(1, _F), cnt)
    h = h0
    for i in range(Wg.shape[0]):
        pT = _sc_aggregate(row, col, _to_sc(g))
        h, g = _tc_layer(_from_sc(pT), h, h0, dv, Wg[i], srelu_bias[i].reshape(1, _F))
    return _tc_out(h, Wout, bout.reshape(1, _NCLS))


# speedup vs baseline: 1.0622x; 1.0622x over previous
import functools

import jax
import jax.numpy as jnp
from jax import lax
from jax.experimental import pallas as pl
from jax.experimental.pallas import tpu as pltpu
from jax.experimental.pallas import tpu_sc as plsc

_N = 10000
_F = 128
_NCLS = 40
_NC = 2
_NS = 16
_NW = _NC * _NS
_LPW = _F // _NW
_E0 = 320000
_EP = 327680
_EC = 2048
_NCHUNK = _EP // _EC
_TRASH = _N
_NPL = 10016
_NFP = _NPL * _LPW

_ALPHA = 0.1
_RW = 0.1
_CS = 0.8

_mesh = plsc.VectorSubcoreMesh(core_axis_name="c", subcore_axis_name="s")


@functools.partial(
    pl.kernel,
    mesh=_mesh,
    compiler_params=pltpu.CompilerParams(needs_layout_passes=False),
    out_type=pltpu.HBM((_NW, _NFP), jnp.float32),
    scratch_types=[
        pltpu.VMEM((_NFP,), jnp.float32),
        pltpu.VMEM((_NFP,), jnp.float32),
        [pltpu.VMEM((_EC,), jnp.int32)] * 2,
        [pltpu.VMEM((_EC,), jnp.int32)] * 2,
        [pltpu.SemaphoreType.DMA] * 2,
        [pltpu.SemaphoreType.DMA] * 2,
        pltpu.SemaphoreType.DMA,
    ],
)
def _sc_aggregate(row_hbm, col_hbm, gt_hbm, out, g_t, acc_t,
                  ridxb, cidxb, rsem, csem, gsem):
    c = lax.axis_index("c")
    s = lax.axis_index("s")
    wid = s * _NC + c

    pltpu.async_copy(gt_hbm.at[wid], g_t, gsem)
    pltpu.async_copy(row_hbm.at[pl.ds(0, _EC)], ridxb[0], rsem[0])
    pltpu.async_copy(col_hbm.at[pl.ds(0, _EC)], cidxb[0], csem[0])

    zero16 = jnp.zeros((16,), jnp.float32)

    def z(i, carry):
        acc_t[pl.ds(i * 16, 16)] = zero16
        return carry

    lax.fori_loop(0, _NFP // 16, z, 0, unroll=4)
    pltpu.make_async_copy(gt_hbm.at[wid], g_t, gsem).wait()

    def pair(k, carry):
        for p in range(2):
            ch = k * 2 + p
            pltpu.make_async_copy(row_hbm.at[pl.ds(0, _EC)], ridxb[p], rsem[p]).wait()
            pltpu.make_async_copy(col_hbm.at[pl.ds(0, _EC)], cidxb[p], csem[p]).wait()

            @pl.when(ch + 1 < _NCHUNK)
            def _():
                nb = (ch + 1) * _EC
                pltpu.async_copy(row_hbm.at[pl.ds(nb, _EC)], ridxb[1 - p], rsem[1 - p])
                pltpu.async_copy(col_hbm.at[pl.ds(nb, _EC)], cidxb[1 - p], csem[1 - p])

            @plsc.parallel_loop(0, _EC // 16, unroll=4)
            def _(i):
                r16 = ridxb[p][pl.ds(i * 16, 16)]
                c16 = cidxb[p][pl.ds(i * 16, 16)]
                for l in range(_LPW):
                    v = plsc.load_gather(g_t, [r16 + (l * _NPL)])
                    plsc.addupdate_scatter(acc_t, [c16 + (l * _NPL)], v)
        return carry

    lax.fori_loop(0, _NCHUNK // 2, pair, 0)
    pltpu.sync_copy(acc_t, out.at[wid])


@functools.partial(
    pl.kernel,
    mesh=_mesh,
    compiler_params=pltpu.CompilerParams(needs_layout_passes=False),
    out_type=pltpu.HBM((_NPL,), jnp.float32),
    scratch_types=[
        pltpu.VMEM((_NPL,), jnp.float32),
        [pltpu.VMEM((_EC,), jnp.int32)] * 2,
        [pltpu.SemaphoreType.DMA] * 2,
    ],
)
def _sc_degree(col_hbm, out, acc_t, cidxb, csem):
    c = lax.axis_index("c")
    s = lax.axis_index("s")
    wid = s * _NC + c
    pltpu.async_copy(col_hbm.at[pl.ds(0, _EC)], cidxb[0], csem[0])
    zero16 = jnp.zeros((16,), jnp.float32)

    def z(i, carry):
        acc_t[pl.ds(i * 16, 16)] = zero16
        return carry

    lax.fori_loop(0, _NPL // 16, z, 0, unroll=4)
    one16 = jnp.ones((16,), jnp.float32)

    def pair(k, carry):
        for p in range(2):
            ch = k * 2 + p

            pltpu.make_async_copy(col_hbm.at[pl.ds(0, _EC)], cidxb[p], csem[p]).wait()

            @pl.when(ch + 1 < _NCHUNK)
            def _():
                nb = (ch + 1) * _EC
                pltpu.async_copy(col_hbm.at[pl.ds(nb, _EC)], cidxb[1 - p], csem[1 - p])

            @plsc.parallel_loop(0, _EC // 16, unroll=4)
            def _(i):
                c16 = cidxb[p][pl.ds(i * 16, 16)]
                plsc.addupdate_scatter(acc_t, [c16], one16)
        return carry

    lax.fori_loop(0, _NCHUNK // 2, pair, 0)

    @pl.when(wid == 0)
    def _():
        pltpu.sync_copy(acc_t, out)


def _to_sc(g):
    gt = g.reshape(_N, _NW, _LPW).transpose(1, 2, 0)
    gt = jnp.pad(gt, ((0, 0), (0, 0), (0, _NPL - _N)))
    return gt.reshape(_NW, _NFP)


def _from_sc(p):
    return p.reshape(_NW, _LPW, _NPL)[:, :, :_N].transpose(2, 0, 1).reshape(_N, _F)


_BLK = 1000


def _tc_init_body(x_ref, win_ref, bin_ref, cnt_ref, h0_ref, g0_ref, dv_ref):
    z = jnp.dot(x_ref[...], win_ref[...], preferred_element_type=jnp.float32)
    h0 = jnp.maximum(z + bin_ref[...], 0.0)
    dv = lax.rsqrt(cnt_ref[...] + 1.0)
    h0_ref[...] = h0
    g0_ref[...] = h0 * dv
    dv_ref[...] = dv


def _tc_init(x, Win, bin_row, cnt):
    return pl.pallas_call(
        _tc_init_body,
        grid=(_N // _BLK,),
        in_specs=[
            pl.BlockSpec((_BLK, _F), lambda i: (i, 0)),
            pl.BlockSpec((_F, _F), lambda i: (0, 0)),
            pl.BlockSpec((1, _F), lambda i: (0, 0)),
            pl.BlockSpec((_BLK, 1), lambda i: (i, 0)),
        ],
        out_specs=[
            pl.BlockSpec((_BLK, _F), lambda i: (i, 0)),
            pl.BlockSpec((_BLK, _F), lambda i: (i, 0)),
            pl.BlockSpec((_BLK, 1), lambda i: (i, 0)),
        ],
        out_shape=[
            jax.ShapeDtypeStruct((_N, _F), jnp.float32),
            jax.ShapeDtypeStruct((_N, _F), jnp.float32),
            jax.ShapeDtypeStruct((_N, 1), jnp.float32),
        ],
    )(x, Win, bin_row, cnt)


def _tc_layer_body(p_ref, h_ref, x0_ref, dv_ref, w_ref, b_ref, hn_ref, gn_ref):
    dv = dv_ref[...]
    h = h_ref[...]
    agg = p_ref[...] * dv + (dv * dv) * h
    h2 = _CS * agg + _RW * h + _ALPHA * x0_ref[...]
    z = jnp.dot(h2, w_ref[...], preferred_element_type=jnp.float32)
    hn = jnp.maximum(z, b_ref[...])
    hn_ref[...] = hn
    gn_ref[...] = hn * dv


def _tc_layer(p, h, x0, dv, W, b_row):
    return pl.pallas_call(
        _tc_layer_body,
        grid=(_N // _BLK,),
        in_specs=[
            pl.BlockSpec((_BLK, _F), lambda i: (i, 0)),
            pl.BlockSpec((_BLK, _F), lambda i: (i, 0)),
            pl.BlockSpec((_BLK, _F), lambda i: (i, 0)),
            pl.BlockSpec((_BLK, 1), lambda i: (i, 0)),
            pl.BlockSpec((_F, _F), lambda i: (0, 0)),
            pl.BlockSpec((1, _F), lambda i: (0, 0)),
        ],
        out_specs=[
            pl.BlockSpec((_BLK, _F), lambda i: (i, 0)),
            pl.BlockSpec((_BLK, _F), lambda i: (i, 0)),
        ],
        out_shape=[
            jax.ShapeDtypeStruct((_N, _F), jnp.float32),
            jax.ShapeDtypeStruct((_N, _F), jnp.float32),
        ],
    )(p, h, x0, dv, W, b_row)


def _tc_out_body(h_ref, w_ref, b_ref, o_ref):
    o_ref[...] = (
        jnp.dot(h_ref[...], w_ref[...], preferred_element_type=jnp.float32)
        + b_ref[...]
    )


def _tc_out(h, Wout, bout_row):
    return pl.pallas_call(
        _tc_out_body,
        grid=(_N // _BLK,),
        in_specs=[
            pl.BlockSpec((_BLK, _F), lambda i: (i, 0)),
            pl.BlockSpec((_F, _NCLS), lambda i: (0, 0)),
            pl.BlockSpec((1, _NCLS), lambda i: (0, 0)),
        ],
        out_specs=pl.BlockSpec((_BLK, _NCLS), lambda i: (i, 0)),
        out_shape=jax.ShapeDtypeStruct((_N, _NCLS), jnp.float32),
    )(h, Wout, bout_row)


def kernel(x, edge_index, Win, bin_, Wg, srelu_bias, Wout, bout):
    npad = _EP - _E0
    row = jnp.concatenate([edge_index[0], jnp.zeros((npad,), jnp.int32)])
    col = jnp.concatenate([edge_index[1], jnp.full((npad,), _TRASH, jnp.int32)])

    cntv = _sc_degree(col)
    cnt = cntv[0:_N].reshape(_N, 1)

    h0, g, dv = _tc_init(x, Win, bin_.reshape(1, _F), cnt)
    h = h0
    for i in range(Wg.shape[0]):
        pT = _sc_aggregate(row, col, _to_sc(g))
        h, g = _tc_layer(_from_sc(pT), h, h0, dv, Wg[i], srelu_bias[i].reshape(1, _F))
    return _tc_out(h, Wout, bout.reshape(1, _NCLS))
